# baseline (speedup 1.0000x reference)
import functools
import math

import jax
import jax.numpy as jnp
from jax import lax
from jax.experimental import pallas as pl
from jax.experimental.pallas import tpu as pltpu
from jax.experimental.pallas import tpu_sc as plsc

_D = 64
_SCALE = math.sqrt(_D)
_NC = 2
_NS = 16
_NW = _NC * _NS
_LANES = 16
_BB = 128


@functools.partial(jax.jit, static_argnames=("b", "l"))
def _embed(xt, table, b, l):
    mesh = plsc.VectorSubcoreMesh(core_axis_name="c", subcore_axis_name="s")

    @functools.partial(
        pl.kernel,
        mesh=mesh,
        compiler_params=pltpu.CompilerParams(
            use_tc_tiling_on_sc=False, needs_layout_passes=False),
        out_type=jax.ShapeDtypeStruct((l, _D, b), jnp.float32),
        scratch_types=[
            pltpu.VMEM((l, _BB), jnp.int32),
            pltpu.VMEM((_BB, _D), jnp.float32),
            pltpu.VMEM((_BB, _D), jnp.float32),
            pltpu.VMEM((_D, _BB), jnp.float32),
            pltpu.VMEM((_D, _BB), jnp.float32),
            pltpu.SemaphoreType.DMA,
            pltpu.SemaphoreType.DMA,
            pltpu.SemaphoreType.DMA,
            pltpu.SemaphoreType.DMA,
        ],
    )
    def k(xt_hbm, table_hbm, out_hbm, idx_v, rows0, rows1, t0, t1,
          g0, g1, s0, s1):
        wid = lax.axis_index("s") * _NC + lax.axis_index("c")
        b0 = pl.multiple_of(wid * _BB, _BB)
        pltpu.sync_copy(xt_hbm.at[:, pl.ds(b0, _BB)], idx_v)

        rows = (rows0, rows1)
        tbuf = (t0, t1)
        gsem = (g0, g1)
        ssem = (s0, s1)
        iota = lax.iota(jnp.int32, _LANES)

        def fire_gather(li, buf):
            return pltpu.async_copy(
                table_hbm.at[idx_v.at[li]], rows[buf], gsem[buf])

        def transpose_add(buf):
            def row_body(r, c2):
                bvec = jnp.full((_LANES,), r, jnp.int32)
                for j in range(_D // _LANES):
                    sl = pl.ds(j * _LANES, _LANES)
                    val = rows[buf][r, sl] + _SCALE
                    plsc.store_scatter(
                        tbuf[buf], [j * _LANES + iota, bvec], val)
                return c2

            lax.fori_loop(0, _BB, row_body, 0, unroll=2)

        gd = {}
        sd = {}
        for li in range(l + 1):
            if li < l:
                buf = li & 1
                if li >= 2:
                    sd[li - 2].wait()
                gd[li] = fire_gather(li, buf)
            if li >= 1:
                pbuf = (li - 1) & 1
                gd[li - 1].wait()
                transpose_add(pbuf)
                sd[li - 1] = pltpu.async_copy(
                    tbuf[pbuf], out_hbm.at[li - 1, :, pl.ds(b0, _BB)],
                    ssem[pbuf])
        sd[l - 2].wait()
        sd[l - 1].wait()

    return k(xt, table)


def kernel(x, table):
    b, l = x.shape
    xt = x.T.astype(jnp.int32)
    out = _embed(xt, table, b, l)
    return jnp.transpose(out, (2, 0, 1))


# speedup vs baseline: 1.2439x; 1.2439x over previous
import functools
import math

import jax
import jax.numpy as jnp
from jax import lax
from jax.experimental import pallas as pl
from jax.experimental.pallas import tpu as pltpu
from jax.experimental.pallas import tpu_sc as plsc

_D = 64
_SCALE = math.sqrt(_D)
_NC = 2
_NS = 16
_NW = _NC * _NS
_LANES = 16
_CHUNK = 800
_SUB = 128


@functools.partial(jax.jit, static_argnames=("n_per_w",))
def _embed(x_flat, table, n_per_w):
    n = x_flat.shape[0]
    n_chunks = n_per_w // _CHUNK
    mesh = plsc.VectorSubcoreMesh(core_axis_name="c", subcore_axis_name="s")

    @functools.partial(
        pl.kernel,
        mesh=mesh,
        compiler_params=pltpu.CompilerParams(
            use_tc_tiling_on_sc=False, needs_layout_passes=False),
        out_type=jax.ShapeDtypeStruct((n // 50, 50, _D), jnp.float32),
        scratch_types=[
            pltpu.VMEM((n_per_w,), jnp.int32),
            pltpu.VMEM((_CHUNK, _D), jnp.float32),
            pltpu.VMEM((_CHUNK, _D), jnp.float32),
            pltpu.SemaphoreType.DMA,
            pltpu.SemaphoreType.DMA,
            pltpu.SemaphoreType.DMA,
            pltpu.SemaphoreType.DMA,
        ],
    )
    def k(x_hbm, table_hbm, out_hbm, idx_v, rows0, rows1, g0, g1, s0, s1):
        wid = lax.axis_index("s") * _NC + lax.axis_index("c")
        base = pl.multiple_of(wid * n_per_w, _CHUNK)
        pltpu.sync_copy(x_hbm.at[pl.ds(base, n_per_w)], idx_v)

        rows = (rows0, rows1)
        gsem = (g0, g1)
        ssem = (s0, s1)

        def fire_gathers(ci, buf):
            descs = []
            for j in range(_CHUNK // _SUB):
                o = ci * _CHUNK + j * _SUB
                descs.append(
                    pltpu.async_copy(
                        table_hbm.at[idx_v.at[pl.ds(o, _SUB)]],
                        rows[buf].at[pl.ds(j * _SUB, _SUB), :],
                        gsem[buf],
                    )
                )
            return descs

        def add_pass(buf):
            def row_body(r, c2):
                for j in range(_D // _LANES):
                    sl = pl.ds(j * _LANES, _LANES)
                    rows[buf][r, sl] = rows[buf][r, sl] + _SCALE
                return c2

            lax.fori_loop(0, _CHUNK, row_body, 0, unroll=4)

        gd = {}
        sd = {}
        for ci in range(n_chunks + 1):
            if ci < n_chunks:
                buf = ci & 1
                if ci >= 2:
                    for d in sd[ci - 2]:
                        d.wait()
                gd[ci] = fire_gathers(ci, buf)
            if ci >= 1:
                pbuf = (ci - 1) & 1
                for d in gd[ci - 1]:
                    d.wait()
                add_pass(pbuf)
                br0 = pl.multiple_of(
                    (base + (ci - 1) * _CHUNK) // 50, _CHUNK // 50)
                ds_list = []
                for kk in range(_CHUNK // 50):
                    ds_list.append(pltpu.async_copy(
                        rows[pbuf].at[pl.ds(kk * 50, 50), :],
                        out_hbm.at[br0 + kk],
                        ssem[pbuf],
                    ))
                sd[ci - 1] = ds_list
        for d in sd[n_chunks - 2]:
            d.wait()
        for d in sd[n_chunks - 1]:
            d.wait()

    return k(x_flat, table)


def kernel(x, table):
    b, l = x.shape
    n = b * l
    n_per_w = n // _NW
    x_flat = x.reshape(n).astype(jnp.int32)
    out = _embed(x_flat, table, n_per_w)
    return out


# speedup vs baseline: 1.2467x; 1.0023x over previous
import functools
import math

import jax
import jax.numpy as jnp
from jax import lax
from jax.experimental import pallas as pl
from jax.experimental.pallas import tpu as pltpu
from jax.experimental.pallas import tpu_sc as plsc

_D = 64
_SCALE = math.sqrt(_D)
_NC = 2
_NS = 16
_NW = _NC * _NS
_LANES = 16
_CHUNK = 640
_SUB = 128


@functools.partial(jax.jit, static_argnames=("n_per_w",))
def _embed(x_flat, table, n_per_w):
    n = x_flat.shape[0]
    n_chunks = n_per_w // _CHUNK
    mesh = plsc.VectorSubcoreMesh(core_axis_name="c", subcore_axis_name="s")

    @functools.partial(
        pl.kernel,
        mesh=mesh,
        compiler_params=pltpu.CompilerParams(
            use_tc_tiling_on_sc=False, needs_layout_passes=False),
        out_type=jax.ShapeDtypeStruct((n, _D), jnp.float32),
        scratch_types=[
            pltpu.VMEM((n_per_w,), jnp.int32),
            pltpu.VMEM((_CHUNK, _D), jnp.float32),
            pltpu.VMEM((_CHUNK, _D), jnp.float32),
            pltpu.SemaphoreType.DMA,
            pltpu.SemaphoreType.DMA,
            pltpu.SemaphoreType.DMA,
            pltpu.SemaphoreType.DMA,
        ],
    )
    def k(x_hbm, table_hbm, out_hbm, idx_v, rows0, rows1, g0, g1, s0, s1):
        wid = lax.axis_index("s") * _NC + lax.axis_index("c")
        base = pl.multiple_of(wid * n_per_w, _CHUNK)
        pltpu.sync_copy(x_hbm.at[pl.ds(base, n_per_w)], idx_v)

        rows = (rows0, rows1)
        gsem = (g0, g1)
        ssem = (s0, s1)

        def fire_gathers(ci, buf):
            descs = []
            for j in range(_CHUNK // _SUB):
                o = ci * _CHUNK + j * _SUB
                descs.append(
                    pltpu.async_copy(
                        table_hbm.at[idx_v.at[pl.ds(o, _SUB)]],
                        rows[buf].at[pl.ds(j * _SUB, _SUB), :],
                        gsem[buf],
                    )
                )
            return descs

        def add_pass(buf):
            def row_body(r, c2):
                for j in range(_D // _LANES):
                    sl = pl.ds(j * _LANES, _LANES)
                    rows[buf][r, sl] = rows[buf][r, sl] + _SCALE
                return c2

            lax.fori_loop(0, _CHUNK, row_body, 0, unroll=4)

        gd = {}
        sd = {}
        for ci in range(n_chunks + 1):
            if ci < n_chunks:
                buf = ci & 1
                if ci >= 2:
                    sd[ci - 2].wait()
                gd[ci] = fire_gathers(ci, buf)
            if ci >= 1:
                pbuf = (ci - 1) & 1
                for d in gd[ci - 1]:
                    d.wait()
                add_pass(pbuf)
                off = pl.multiple_of(base + (ci - 1) * _CHUNK, _CHUNK)
                sd[ci - 1] = pltpu.async_copy(
                    rows[pbuf], out_hbm.at[pl.ds(off, _CHUNK)], ssem[pbuf]
                )
        sd[n_chunks - 2].wait()
        sd[n_chunks - 1].wait()

    return k(x_flat, table)


def kernel(x, table):
    b, l = x.shape
    n = b * l
    n_per_w = n // _NW
    x_flat = x.reshape(n).astype(jnp.int32)
    out = _embed(x_flat, table, n_per_w)
    return out.reshape(b, l, _D)
